# baseline (speedup 1.0000x reference)
import functools

import jax
import jax.numpy as jnp
from jax import lax
from jax.experimental import pallas as pl
from jax.experimental.pallas import tpu as pltpu
from jax.experimental.pallas import tpu_sc as plsc

N = 100000
M = 64
NMOL = 1000
NMOL_PAD = 1008
NUM_PAD = 100008
CHUNK = 32
CW = CHUNK * M
NCHUNKS = N // CHUNK
NW = 32

_mesh = plsc.VectorSubcoreMesh(core_axis_name="c", subcore_axis_name="s")


def _srrep_body(num_hbm, pf_hbm, nb_hbm, d_hbm, mol_hbm, out_hbm,
                num_v, pf_v, A_v, Z_v, A16_v, Z16_v, stage_v,
                nb_v, d_v, mol_v, macc_v, sem0, sem1):
    c = lax.axis_index("c")
    s = lax.axis_index("s")
    wid = s * 2 + c
    iota = lax.iota(jnp.int32, 16)
    zeros16 = jnp.zeros((16,), jnp.float32)
    zeros16i = jnp.zeros((16,), jnp.int32)

    pltpu.sync_copy(num_hbm, num_v)
    pltpu.sync_copy(pf_hbm, pf_v)
    for k in range(6):
        base = iota + 16 * k
        a = plsc.load_gather(pf_v, [base * 2])
        z = plsc.load_gather(pf_v, [base * 2 + 1])
        plsc.store_scatter(A_v, [base], a)
        plsc.store_scatter(Z_v, [base], z)

    def rep_body(si, carry):
        av = plsc.load_gather(A_v, [zeros16i + si])
        zv = plsc.load_gather(Z_v, [zeros16i + si])
        plsc.store_scatter(A16_v, [si * 16 + iota], av)
        plsc.store_scatter(Z16_v, [si * 16 + iota], zv)
        return carry

    lax.fori_loop(0, 87, rep_body, jnp.int32(0))

    def zero_body(r, carry):
        row = zeros16i + r
        for v in range(NMOL_PAD // 16):
            plsc.store_scatter(macc_v, [row, v * 16 + iota], zeros16)
        return carry

    lax.fori_loop(0, 16, zero_body, jnp.int32(0))

    start = jnp.minimum(98 * wid, 97 * wid + 21)
    end = jnp.minimum(98 * (wid + 1), 97 * (wid + 1) + 21)

    def issue(ci, par):
        sem = sem0 if par == 0 else sem1
        voff = par * CW
        a0 = ci * CHUNK
        pltpu.async_copy(nb_hbm.at[pl.ds(a0 * M, CW)],
                         nb_v.at[pl.ds(voff, CW)], sem)
        pltpu.async_copy(d_hbm.at[pl.ds(a0 * M, CW)],
                         d_v.at[pl.ds(voff, CW)], sem)
        pltpu.async_copy(mol_hbm.at[pl.ds(a0, CHUNK)],
                         mol_v.at[pl.ds(par * CHUNK, CHUNK)], sem)

    def drain(par):
        sem = sem0 if par == 0 else sem1
        voff = par * CW
        pltpu.make_async_copy(nb_hbm.at[pl.ds(0, CW)],
                              nb_v.at[pl.ds(voff, CW)], sem).wait()
        pltpu.make_async_copy(d_hbm.at[pl.ds(0, CW)],
                              d_v.at[pl.ds(voff, CW)], sem).wait()
        pltpu.make_async_copy(mol_hbm.at[pl.ds(0, CHUNK)],
                              mol_v.at[pl.ds(par * CHUNK, CHUNK)], sem).wait()

    issue(start, 0)
    issue(start + 1, 1)

    def atom_body(wb, g0b, a):
        si_b = plsc.load_gather(num_v, [zeros16i + (g0b + a)])
        na_b = -plsc.load_gather(A_v, [si_b])
        sb = wb + a * M

        for q in range(4):
            iv = sb + q * 16 + iota
            nb = plsc.load_gather(nb_v, [iv])
            dd = plsc.load_gather(d_v, [iv])
            sj = plsc.load_gather(num_v, [nb])
            aj = plsc.load_gather(A16_v, [sj * 16 + iota])
            zj = plsc.load_gather(Z16_v, [sj * 16 + iota])
            ih = jnp.int32(0x5F3759DF) - (plsc.bitcast(dd, jnp.int32) >> 1)
            r = plsc.bitcast(ih, jnp.float32)
            hd = 0.5 * dd
            r = r * (1.5 - hd * r * r)
            r = r * (1.5 - hd * r * r)
            t = (na_b * aj) * (dd * dd * r)
            e = jnp.exp(t) * (zj * (r * r))
            if q == 0:
                plsc.store_scatter(stage_v, [iota * 17 + a], e)
            else:
                plsc.addupdate_scatter(stage_v, [iota * 17 + a], e)

    def block(g0, loff, b):
        g0b = g0 + b * 16
        wb = loff + b * 16 * M

        def a_body(a, carry):
            atom_body(wb, g0b, a)
            return carry

        lax.fori_loop(0, 16, a_body, jnp.int32(0))

        e16 = plsc.load_gather(stage_v, [iota])
        for rr in range(1, 16):
            e16 = e16 + plsc.load_gather(stage_v, [rr * 17 + iota])
        gi = g0b + iota
        s_i = plsc.load_gather(num_v, [gi])
        z_i = plsc.load_gather(Z_v, [s_i])
        mol = plsc.load_gather(mol_v, [loff // M + b * 16 + iota])
        plsc.addupdate_scatter(macc_v, [iota, mol], e16 * z_i)

    def chunk_body(ci, carry):
        p = (ci - start) & 1
        g0 = ci * CHUNK
        loff = p * CW

        @pl.when(p == 0)
        def _w0():
            drain(0)

        @pl.when(p == 1)
        def _w1():
            drain(1)

        for b in range(CHUNK // 16):
            block(g0, loff, b)

        nxt = ci + 2

        @pl.when((p == 0) & (nxt < end))
        def _i0():
            issue(nxt, 0)

        @pl.when((p == 1) & (nxt < end))
        def _i1():
            issue(nxt, 1)

        return carry

    lax.fori_loop(start, end, chunk_body, jnp.int32(0))

    def fold_body(i, carry):
        row = zeros16i + i
        for v in range(NMOL_PAD // 16):
            iv = v * 16 + iota
            t = plsc.load_gather(macc_v, [row, iv])
            t0 = plsc.load_gather(macc_v, [zeros16i, iv])
            plsc.store_scatter(macc_v, [zeros16i, iv], t0 + t)
        return carry

    lax.fori_loop(1, 16, fold_body, jnp.int32(0))
    pltpu.sync_copy(macc_v.at[0], out_hbm.at[wid])


_srrep = functools.partial(
    pl.kernel,
    out_type=jax.ShapeDtypeStruct((NW, NMOL_PAD), jnp.float32),
    mesh=_mesh,
    compiler_params=pltpu.CompilerParams(needs_layout_passes=False),
    scratch_types=[
        pltpu.VMEM((NUM_PAD,), jnp.int32),
        pltpu.VMEM((192,), jnp.float32),
        pltpu.VMEM((96,), jnp.float32),
        pltpu.VMEM((96,), jnp.float32),
        pltpu.VMEM((87 * 16,), jnp.float32),
        pltpu.VMEM((87 * 16,), jnp.float32),
        pltpu.VMEM((16 * 17,), jnp.float32),
        pltpu.VMEM((2 * CW,), jnp.int32),
        pltpu.VMEM((2 * CW,), jnp.float32),
        pltpu.VMEM((2 * CHUNK,), jnp.int32),
        pltpu.VMEM((16, NMOL_PAD), jnp.float32),
        pltpu.SemaphoreType.DMA,
        pltpu.SemaphoreType.DMA,
    ],
)(_srrep_body)


def kernel(numbers, nbmat, d_ij, mol_idx, params):
    numbers = numbers.astype(jnp.int32)
    num_pad = jnp.concatenate(
        [numbers, jnp.zeros((NUM_PAD - (N + 1),), jnp.int32)])
    pf = jnp.concatenate(
        [params.reshape(-1).astype(jnp.float32),
         jnp.zeros((192 - 2 * 87,), jnp.float32)])
    nb_flat = nbmat.astype(jnp.int32).reshape(-1)
    d_flat = d_ij.astype(jnp.float32).reshape(-1)
    mol = mol_idx.astype(jnp.int32)
    out2 = _srrep(num_pad, pf, nb_flat, d_flat, mol)
    return out2.sum(0)[:NMOL]


# speedup vs baseline: 1.3174x; 1.3174x over previous
import functools

import jax
import jax.numpy as jnp
from jax import lax
from jax.experimental import pallas as pl
from jax.experimental.pallas import tpu as pltpu
from jax.experimental.pallas import tpu_sc as plsc

N = 100000
M = 64
NMOL = 1000
NMOL_PAD = 1008
NUM_PAD = 100008
CHUNK = 48
CW = CHUNK * M
FULL_CHUNKS = N // CHUNK
TAIL_ATOMS = N - FULL_CHUNKS * CHUNK
NW = 32

_mesh = plsc.VectorSubcoreMesh(core_axis_name="c", subcore_axis_name="s")


def _srrep_body(num_hbm, pf_hbm, nb_hbm, d_hbm, mol_hbm, out_hbm,
                num_v, pf_v, A_v, Z_v, nb_v, d_v, mol_v, macc_v,
                sem0, sem1):
    c = lax.axis_index("c")
    s = lax.axis_index("s")
    wid = s * 2 + c
    iota = lax.iota(jnp.int32, 16)

    pltpu.sync_copy(num_hbm, num_v)
    pltpu.sync_copy(pf_hbm, pf_v)
    for k in range(6):
        base = iota + 16 * k
        a = plsc.load_gather(pf_v, [base * 2])
        z = plsc.load_gather(pf_v, [base * 2 + 1])
        plsc.store_scatter(A_v, [base], a)
        plsc.store_scatter(Z_v, [base], z)
    zeros16 = jnp.zeros((16,), jnp.float32)
    zeros16i = jnp.zeros((16,), jnp.int32)

    def zero_body(r, carry):
        row = zeros16i + r
        for v in range(NMOL_PAD // 16):
            plsc.store_scatter(macc_v, [row, v * 16 + iota], zeros16)
        return carry

    lax.fori_loop(0, 16, zero_body, jnp.int32(0))

    def block(g0, loff, b):
        gi = g0 + b * 16 + iota
        s_i = plsc.load_gather(num_v, [gi])
        a_i = plsc.load_gather(A_v, [s_i])
        z_i = plsc.load_gather(Z_v, [s_i])
        na_i = -a_i
        mol = plsc.load_gather(mol_v, [loff // M + b * 16 + iota])
        iv0 = loff + b * 16 * M + iota * M

        def m_body(m, acc):
            iv = iv0 + m
            nb = plsc.load_gather(nb_v, [iv])
            dd = plsc.load_gather(d_v, [iv])
            sj = plsc.load_gather(num_v, [nb])
            aj = plsc.load_gather(A_v, [sj])
            zj = plsc.load_gather(Z_v, [sj])
            ih = jnp.int32(0x5F3759DF) - (plsc.bitcast(dd, jnp.int32) >> 1)
            r = plsc.bitcast(ih, jnp.float32)
            hd = 0.5 * dd
            r = r * (1.5 - hd * r * r)
            r = r * (1.5 - hd * r * r)
            t = (na_i * aj) * (dd * dd * r)
            e = jnp.exp(t) * (zj * (r * r))
            return acc + e

        acc = lax.fori_loop(0, M, m_body, jnp.zeros((16,), jnp.float32),
                            unroll=4)
        plsc.addupdate_scatter(macc_v, [iota, mol], acc * z_i)

    start = jnp.minimum(66 * wid, 65 * wid + 3)
    end = jnp.minimum(66 * (wid + 1), 65 * (wid + 1) + 3)

    def issue(ci, par):
        sem = sem0 if par == 0 else sem1
        voff = par * CW
        a0 = ci * CHUNK
        pltpu.async_copy(nb_hbm.at[pl.ds(a0 * M, CW)],
                         nb_v.at[pl.ds(voff, CW)], sem)
        pltpu.async_copy(d_hbm.at[pl.ds(a0 * M, CW)],
                         d_v.at[pl.ds(voff, CW)], sem)
        pltpu.async_copy(mol_hbm.at[pl.ds(a0, CHUNK)],
                         mol_v.at[pl.ds(par * CHUNK, CHUNK)], sem)

    def drain(par):
        sem = sem0 if par == 0 else sem1
        voff = par * CW
        pltpu.make_async_copy(nb_hbm.at[pl.ds(0, CW)],
                              nb_v.at[pl.ds(voff, CW)], sem).wait()
        pltpu.make_async_copy(d_hbm.at[pl.ds(0, CW)],
                              d_v.at[pl.ds(voff, CW)], sem).wait()
        pltpu.make_async_copy(mol_hbm.at[pl.ds(0, CHUNK)],
                              mol_v.at[pl.ds(par * CHUNK, CHUNK)], sem).wait()

    issue(start, 0)
    issue(start + 1, 1)

    def chunk_body(ci, carry):
        p = (ci - start) & 1
        g0 = ci * CHUNK
        loff = p * CW

        @pl.when(p == 0)
        def _w0():
            drain(0)

        @pl.when(p == 1)
        def _w1():
            drain(1)

        for b in range(CHUNK // 16):
            block(g0, loff, b)

        nxt = ci + 2

        @pl.when((p == 0) & (nxt < end))
        def _i0():
            issue(nxt, 0)

        @pl.when((p == 1) & (nxt < end))
        def _i1():
            issue(nxt, 1)

        return carry

    lax.fori_loop(start, end, chunk_body, jnp.int32(0))

    @pl.when(wid == NW - 1)
    def _tail():
        a0 = FULL_CHUNKS * CHUNK
        pltpu.sync_copy(nb_hbm.at[pl.ds(a0 * M, TAIL_ATOMS * M)],
                        nb_v.at[pl.ds(0, TAIL_ATOMS * M)])
        pltpu.sync_copy(d_hbm.at[pl.ds(a0 * M, TAIL_ATOMS * M)],
                        d_v.at[pl.ds(0, TAIL_ATOMS * M)])
        pltpu.sync_copy(mol_hbm.at[pl.ds(a0, TAIL_ATOMS)],
                        mol_v.at[pl.ds(0, TAIL_ATOMS)])
        block(jnp.int32(a0), jnp.int32(0), 0)

    def fold_body(i, carry):
        row = zeros16i + i
        for v in range(NMOL_PAD // 16):
            iv = v * 16 + iota
            t = plsc.load_gather(macc_v, [row, iv])
            t0 = plsc.load_gather(macc_v, [zeros16i, iv])
            plsc.store_scatter(macc_v, [zeros16i, iv], t0 + t)
        return carry

    lax.fori_loop(1, 16, fold_body, jnp.int32(0))
    pltpu.sync_copy(macc_v.at[0], out_hbm.at[wid])


_srrep = functools.partial(
    pl.kernel,
    out_type=jax.ShapeDtypeStruct((NW, NMOL_PAD), jnp.float32),
    mesh=_mesh,
    compiler_params=pltpu.CompilerParams(needs_layout_passes=False),
    scratch_types=[
        pltpu.VMEM((NUM_PAD,), jnp.int32),
        pltpu.VMEM((192,), jnp.float32),
        pltpu.VMEM((96,), jnp.float32),
        pltpu.VMEM((96,), jnp.float32),
        pltpu.VMEM((2 * CW,), jnp.int32),
        pltpu.VMEM((2 * CW,), jnp.float32),
        pltpu.VMEM((2 * CHUNK,), jnp.int32),
        pltpu.VMEM((16, NMOL_PAD), jnp.float32),
        pltpu.SemaphoreType.DMA,
        pltpu.SemaphoreType.DMA,
    ],
)(_srrep_body)


def kernel(numbers, nbmat, d_ij, mol_idx, params):
    numbers = numbers.astype(jnp.int32)
    num_pad = jnp.concatenate(
        [numbers, jnp.zeros((NUM_PAD - (N + 1),), jnp.int32)])
    pf = jnp.concatenate(
        [params.reshape(-1).astype(jnp.float32),
         jnp.zeros((192 - 2 * 87,), jnp.float32)])
    nb_flat = nbmat.astype(jnp.int32).reshape(-1)
    d_flat = d_ij.astype(jnp.float32).reshape(-1)
    mol = mol_idx.astype(jnp.int32)
    out2 = _srrep(num_pad, pf, nb_flat, d_flat, mol)
    return out2.sum(0)[:NMOL]


# speedup vs baseline: 1.3209x; 1.0027x over previous
import functools

import jax
import jax.numpy as jnp
from jax import lax
from jax.experimental import pallas as pl
from jax.experimental.pallas import tpu as pltpu
from jax.experimental.pallas import tpu_sc as plsc

N = 100000
M = 64
NMOL = 1000
NMOL_PAD = 1008
NUM_PAD = 100008
CHUNK = 48
CW = CHUNK * M
FULL_CHUNKS = N // CHUNK
TAIL_ATOMS = N - FULL_CHUNKS * CHUNK
NW = 32

_mesh = plsc.VectorSubcoreMesh(core_axis_name="c", subcore_axis_name="s")


def _srrep_body(num_hbm, pf_hbm, nb_hbm, d_hbm, mol_hbm, out_hbm,
                num_v, pf_v, A_v, Z_v, nb_v, d_v, mol_v, macc_v,
                sem0, sem1):
    c = lax.axis_index("c")
    s = lax.axis_index("s")
    wid = s * 2 + c
    iota = lax.iota(jnp.int32, 16)

    pltpu.sync_copy(num_hbm, num_v)
    pltpu.sync_copy(pf_hbm, pf_v)
    for k in range(6):
        base = iota + 16 * k
        a = plsc.load_gather(pf_v, [base * 2])
        z = plsc.load_gather(pf_v, [base * 2 + 1])
        plsc.store_scatter(A_v, [base], a)
        plsc.store_scatter(Z_v, [base], z)
    zeros16 = jnp.zeros((16,), jnp.float32)
    zeros16i = jnp.zeros((16,), jnp.int32)

    def zero_body(r, carry):
        row = zeros16i + r
        for v in range(NMOL_PAD // 16):
            plsc.store_scatter(macc_v, [row, v * 16 + iota], zeros16)
        return carry

    lax.fori_loop(0, 16, zero_body, jnp.int32(0))

    def block(g0, loff, b):
        gi = g0 + b * 16 + iota
        s_i = plsc.load_gather(num_v, [gi])
        a_i = plsc.load_gather(A_v, [s_i])
        z_i = plsc.load_gather(Z_v, [s_i])
        na_i = -a_i
        mol = plsc.load_gather(mol_v, [loff // M + b * 16 + iota])
        iv0 = loff + b * 16 * M + iota * M

        def m_body(m, acc):
            iv = iv0 + m
            nb = plsc.load_gather(nb_v, [iv])
            dd = plsc.load_gather(d_v, [iv])
            sj = plsc.load_gather(num_v, [nb])
            aj = plsc.load_gather(A_v, [sj])
            zj = plsc.load_gather(Z_v, [sj])
            ih = jnp.int32(0x5F3759DF) - (plsc.bitcast(dd, jnp.int32) >> 1)
            r = plsc.bitcast(ih, jnp.float32)
            hd = 0.5 * dd
            r = r * (1.5 - hd * r * r)
            r = r * (1.5 - hd * r * r)
            t = (na_i * aj) * (dd * dd * r)
            e = jnp.exp(t) * (zj * (r * r))
            return acc + e

        acc = lax.fori_loop(0, M, m_body, jnp.zeros((16,), jnp.float32))
        plsc.addupdate_scatter(macc_v, [iota, mol], acc * z_i)

    start = jnp.minimum(66 * wid, 65 * wid + 3)
    end = jnp.minimum(66 * (wid + 1), 65 * (wid + 1) + 3)

    def issue(ci, par):
        sem = sem0 if par == 0 else sem1
        voff = par * CW
        a0 = ci * CHUNK
        pltpu.async_copy(nb_hbm.at[pl.ds(a0 * M, CW)],
                         nb_v.at[pl.ds(voff, CW)], sem)
        pltpu.async_copy(d_hbm.at[pl.ds(a0 * M, CW)],
                         d_v.at[pl.ds(voff, CW)], sem)
        pltpu.async_copy(mol_hbm.at[pl.ds(a0, CHUNK)],
                         mol_v.at[pl.ds(par * CHUNK, CHUNK)], sem)

    def drain(par):
        sem = sem0 if par == 0 else sem1
        voff = par * CW
        pltpu.make_async_copy(nb_hbm.at[pl.ds(0, CW)],
                              nb_v.at[pl.ds(voff, CW)], sem).wait()
        pltpu.make_async_copy(d_hbm.at[pl.ds(0, CW)],
                              d_v.at[pl.ds(voff, CW)], sem).wait()
        pltpu.make_async_copy(mol_hbm.at[pl.ds(0, CHUNK)],
                              mol_v.at[pl.ds(par * CHUNK, CHUNK)], sem).wait()

    issue(start, 0)
    issue(start + 1, 1)

    def chunk_body(ci, carry):
        p = (ci - start) & 1
        g0 = ci * CHUNK
        loff = p * CW

        @pl.when(p == 0)
        def _w0():
            drain(0)

        @pl.when(p == 1)
        def _w1():
            drain(1)

        for b in range(CHUNK // 16):
            block(g0, loff, b)

        nxt = ci + 2

        @pl.when((p == 0) & (nxt < end))
        def _i0():
            issue(nxt, 0)

        @pl.when((p == 1) & (nxt < end))
        def _i1():
            issue(nxt, 1)

        return carry

    lax.fori_loop(start, end, chunk_body, jnp.int32(0))

    @pl.when(wid == NW - 1)
    def _tail():
        a0 = FULL_CHUNKS * CHUNK
        pltpu.sync_copy(nb_hbm.at[pl.ds(a0 * M, TAIL_ATOMS * M)],
                        nb_v.at[pl.ds(0, TAIL_ATOMS * M)])
        pltpu.sync_copy(d_hbm.at[pl.ds(a0 * M, TAIL_ATOMS * M)],
                        d_v.at[pl.ds(0, TAIL_ATOMS * M)])
        pltpu.sync_copy(mol_hbm.at[pl.ds(a0, TAIL_ATOMS)],
                        mol_v.at[pl.ds(0, TAIL_ATOMS)])
        block(jnp.int32(a0), jnp.int32(0), 0)

    def fold_body(i, carry):
        row = zeros16i + i
        for v in range(NMOL_PAD // 16):
            iv = v * 16 + iota
            t = plsc.load_gather(macc_v, [row, iv])
            t0 = plsc.load_gather(macc_v, [zeros16i, iv])
            plsc.store_scatter(macc_v, [zeros16i, iv], t0 + t)
        return carry

    lax.fori_loop(1, 16, fold_body, jnp.int32(0))
    pltpu.sync_copy(macc_v.at[0], out_hbm.at[wid])


_srrep = functools.partial(
    pl.kernel,
    out_type=jax.ShapeDtypeStruct((NW, NMOL_PAD), jnp.float32),
    mesh=_mesh,
    compiler_params=pltpu.CompilerParams(needs_layout_passes=False),
    scratch_types=[
        pltpu.VMEM((NUM_PAD,), jnp.int32),
        pltpu.VMEM((192,), jnp.float32),
        pltpu.VMEM((96,), jnp.float32),
        pltpu.VMEM((96,), jnp.float32),
        pltpu.VMEM((2 * CW,), jnp.int32),
        pltpu.VMEM((2 * CW,), jnp.float32),
        pltpu.VMEM((2 * CHUNK,), jnp.int32),
        pltpu.VMEM((16, NMOL_PAD), jnp.float32),
        pltpu.SemaphoreType.DMA,
        pltpu.SemaphoreType.DMA,
    ],
)(_srrep_body)


def _combine_body(x_ref, o_ref):
    o_ref[...] = jnp.sum(x_ref[...], axis=0, keepdims=True)


_combine = pl.pallas_call(
    _combine_body,
    out_shape=jax.ShapeDtypeStruct((1, NMOL_PAD), jnp.float32),
)


def kernel(numbers, nbmat, d_ij, mol_idx, params):
    numbers = numbers.astype(jnp.int32)
    num_pad = jnp.concatenate(
        [numbers, jnp.zeros((NUM_PAD - (N + 1),), jnp.int32)])
    pf = jnp.concatenate(
        [params.reshape(-1).astype(jnp.float32),
         jnp.zeros((192 - 2 * 87,), jnp.float32)])
    nb_flat = nbmat.astype(jnp.int32).reshape(-1)
    d_flat = d_ij.astype(jnp.float32).reshape(-1)
    mol = mol_idx.astype(jnp.int32)
    out2 = _srrep(num_pad, pf, nb_flat, d_flat, mol)
    return _combine(out2)[0, :NMOL]
